# baseline (speedup 1.0000x reference)
import functools

import jax
import jax.numpy as jnp
from jax import lax
from jax.experimental import pallas as pl
from jax.experimental.pallas import tpu as pltpu
from jax.experimental.pallas import tpu_sc as plsc

B, N, L, D = 1, 512, 1024, 64
V = 21
TOK = B * N * L
NC, NS = 2, 16
NW = NC * NS
CHUNK = 256
CH_PER_W = TOK // (NW * CHUNK)
PAIRS = CH_PER_W // 2
IDX_ROWS = CHUNK // 128
Q_PER_ROW = L // CHUNK
STAGE = CHUNK * D
N_STRIDE = D * L
DG_STRIDE = 8 * L
PIECE = 2 * 8 * 128


def _table_body(we_ref, wq_ref, pe_ref, out_ref):
    pe = pe_ref[...]
    we = we_ref[...]
    for q in range(2):
        wq = wq_ref[q]
        out_ref[q] = pe[:, None, :] + we[None, :, :] + wq[None, None, :]


def _build_table(W_emb, W_q, pos_enc):
    t = pl.pallas_call(
        _table_body,
        out_shape=jax.ShapeDtypeStruct((2, L, V, D), jnp.float32),
    )(W_emb, W_q, pos_enc)
    return t.reshape(2 * L * V, D)


def _gather_kernel(
    table_hbm, x_hbm, out_hbm,
    x_v0, x_v1, idx_v0, idx_v1, rows_v0, rows_v1, stg0, stg1,
    gsem0, gsem1, ssem0, ssem1,
):
    wid = lax.axis_index("s") * NC + lax.axis_index("c")
    iota16 = lax.iota(jnp.int32, 16)

    def prepare(g, x_v, idx_v, rows_v, gsem):
        base = g * CHUNK
        pltpu.sync_copy(x_hbm.at[pl.ds(base, CHUNK)], x_v)
        l_base = (g % Q_PER_ROW) * CHUNK
        qoff = jnp.where(g >= Q_PER_ROW, L * V, 0).astype(jnp.int32)

        for j in range(IDX_ROWS):
            def idx_body(i, _):
                t = j * 128 + i * 16
                xv = x_v[pl.ds(t, 16)]
                lv = iota16 + (l_base + t)
                idx_v[j, pl.ds(i * 16, 16)] = xv + lv * V + qoff
                return 0

            lax.fori_loop(0, 128 // 16, idx_body, 0)

        return [
            pltpu.async_copy(
                table_hbm.at[idx_v.at[j]],
                rows_v.at[pl.ds(j * 128, 128)],
                gsem,
            )
            for j in range(IDX_ROWS)
        ]

    w_pats = []
    s_pats = []
    for s in range(16):
        w = (iota16 + s) & 15
        w_pats.append(w)
        s_pats.append(((w >> 3) << 11) + ((w & 7) << 7) + iota16)

    def transpose_and_emit(g, rows_v, stg, ssem):
        n = g // Q_PER_ROW
        lq = g % Q_PER_ROW

        def tb_body(tb, _):
            t0 = tb * 16
            rowidx = iota16 + t0
            tconst = (t0 // 128) * 1024 + (t0 % 128)
            for jd in range(4):
                d0 = jd * 16
                bsplat = jnp.zeros((16,), jnp.int32) + (tconst + (d0 >> 3) * PIECE)
                for s in range(16):
                    v = plsc.load_gather(rows_v, [rowidx, w_pats[s] + d0])
                    plsc.store_scatter(stg, [s_pats[s] + bsplat], v)
            return 0

        lax.fori_loop(0, CHUNK // 16, tb_body, 0)

        out_base = n * N_STRIDE + lq * PIECE
        return [
            pltpu.async_copy(
                stg.at[pl.ds(dg * PIECE, PIECE)],
                out_hbm.at[pl.ds(out_base + dg * DG_STRIDE, PIECE)],
                ssem,
            )
            for dg in range(8)
        ]

    def drain_stage(g, stg, ssem):
        n = g // Q_PER_ROW
        lq = g % Q_PER_ROW
        out_base = n * N_STRIDE + lq * PIECE
        for dg in range(8):
            pltpu.make_async_copy(
                stg.at[pl.ds(dg * PIECE, PIECE)],
                out_hbm.at[pl.ds(out_base + dg * DG_STRIDE, PIECE)],
                ssem,
            ).wait()

    def pair_body(p, carry):
        g0 = wid * CH_PER_W + 2 * p
        g1 = g0 + 1

        cps0 = prepare(g0, x_v0, idx_v0, rows_v0, gsem0)
        cps1 = prepare(g1, x_v1, idx_v1, rows_v1, gsem1)

        for cp in cps0:
            cp.wait()

        @pl.when(p > 0)
        def _():
            drain_stage(g0, stg0, ssem0)

        transpose_and_emit(g0, rows_v0, stg0, ssem0)

        for cp in cps1:
            cp.wait()

        @pl.when(p > 0)
        def _():
            drain_stage(g1, stg1, ssem1)

        transpose_and_emit(g1, rows_v1, stg1, ssem1)
        return carry

    lax.fori_loop(0, PAIRS, pair_body, 0)

    g_last0 = wid * CH_PER_W + CH_PER_W - 2
    g_last1 = wid * CH_PER_W + CH_PER_W - 1
    drain_stage(g_last0, stg0, ssem0)
    drain_stage(g_last1, stg1, ssem1)


def _gather(table, x_flat):
    mesh = plsc.VectorSubcoreMesh(core_axis_name="c", subcore_axis_name="s")
    k = functools.partial(
        pl.kernel,
        mesh=mesh,
        out_type=jax.ShapeDtypeStruct((TOK * D,), jnp.float32),
        scratch_types=[
            pltpu.VMEM((CHUNK,), jnp.int32),
            pltpu.VMEM((CHUNK,), jnp.int32),
            pltpu.VMEM((IDX_ROWS, 128), jnp.int32),
            pltpu.VMEM((IDX_ROWS, 128), jnp.int32),
            pltpu.VMEM((CHUNK, D), jnp.float32),
            pltpu.VMEM((CHUNK, D), jnp.float32),
            pltpu.VMEM((STAGE,), jnp.float32),
            pltpu.VMEM((STAGE,), jnp.float32),
            pltpu.SemaphoreType.DMA,
            pltpu.SemaphoreType.DMA,
            pltpu.SemaphoreType.DMA,
            pltpu.SemaphoreType.DMA,
        ],
        compiler_params=pltpu.CompilerParams(
            use_tc_tiling_on_sc=False, needs_layout_passes=False
        ),
    )(_gather_kernel)
    return k(table, x_flat)


def kernel(x, W_emb, W_q, pos_enc):
    table = _build_table(W_emb, W_q, pos_enc)
    x_flat = x.reshape(TOK).astype(jnp.int32)
    out1 = _gather(table, x_flat)
    out6 = out1.reshape(B, N, 8, L // 128, 8, 128)
    return out6.transpose(0, 1, 3, 5, 2, 4).reshape(B, N, L, D)


# speedup vs baseline: 1.8549x; 1.8549x over previous
import functools

import jax
import jax.numpy as jnp
from jax import lax
from jax.experimental import pallas as pl
from jax.experimental.pallas import tpu as pltpu
from jax.experimental.pallas import tpu_sc as plsc

B, N, L, D = 1, 512, 1024, 64
V = 21
TOK = B * N * L
NC, NS = 2, 16
NW = NC * NS
CHUNK = 256
CH_PER_W = TOK // (NW * CHUNK)
PAIRS = CH_PER_W // 2
IDX_ROWS = CHUNK // 128
Q_PER_ROW = L // CHUNK
STAGE = CHUNK * D
N_STRIDE = D * L
DG_STRIDE = 8 * L
PIECE = 2 * 8 * 128


def _table_body(we_ref, wq_ref, pe_ref, out_ref):
    pe = pe_ref[...]
    we = we_ref[...]
    for q in range(2):
        wq = wq_ref[q]
        out_ref[q] = pe[:, None, :] + we[None, :, :] + wq[None, None, :]


def _build_table(W_emb, W_q, pos_enc):
    t = pl.pallas_call(
        _table_body,
        out_shape=jax.ShapeDtypeStruct((2, L, V, D), jnp.float32),
    )(W_emb, W_q, pos_enc)
    return t.reshape(2 * L * V, D)


def _gather_kernel(
    table_hbm, x_hbm, out_hbm,
    x_v0, x_v1, idx_v0, idx_v1, rows_v0, rows_v1, stg0, stg1,
    gsem0, gsem1, ssem0, ssem1,
):
    wid = lax.axis_index("s") * NC + lax.axis_index("c")
    iota16 = lax.iota(jnp.int32, 16)

    def prepare(g, x_v, idx_v, rows_v, gsem):
        base = g * CHUNK
        pltpu.sync_copy(x_hbm.at[pl.ds(base, CHUNK)], x_v)
        l_base = (g % Q_PER_ROW) * CHUNK
        qoff = jnp.where(g >= Q_PER_ROW, L * V, 0).astype(jnp.int32)

        for j in range(IDX_ROWS):
            def idx_body(i, _):
                t = j * 128 + i * 16
                xv = x_v[pl.ds(t, 16)]
                lv = iota16 + (l_base + t)
                idx_v[j, pl.ds(i * 16, 16)] = xv + lv * V + qoff
                return 0

            lax.fori_loop(0, 128 // 16, idx_body, 0)

        return [
            pltpu.async_copy(
                table_hbm.at[idx_v.at[j]],
                rows_v.at[pl.ds(j * 128, 128)],
                gsem,
            )
            for j in range(IDX_ROWS)
        ]

    w_pats = []
    s_pats = []
    for s in range(16):
        w = (iota16 + s) & 15
        w_pats.append(w)
        s_pats.append(((w >> 3) << 11) + ((w & 7) << 7) + iota16)

    def transpose_and_emit(g, rows_v, stg, ssem):
        n = g // Q_PER_ROW
        lq = g % Q_PER_ROW

        def tb_body(tb, _):
            t0 = tb * 16
            rowidx = iota16 + t0
            tconst = (t0 // 128) * 1024 + (t0 % 128)
            for jd in range(4):
                d0 = jd * 16
                bsplat = jnp.zeros((16,), jnp.int32) + (tconst + (d0 >> 3) * PIECE)
                for h in range(2):
                    vals = [
                        plsc.load_gather(rows_v, [rowidx, w_pats[h * 8 + s] + d0])
                        for s in range(8)
                    ]
                    for s in range(8):
                        plsc.store_scatter(stg, [s_pats[h * 8 + s] + bsplat], vals[s])
            return 0

        lax.fori_loop(0, CHUNK // 16, tb_body, 0)

        out_base = n * N_STRIDE + lq * PIECE
        return [
            pltpu.async_copy(
                stg.at[pl.ds(dg * PIECE, PIECE)],
                out_hbm.at[pl.ds(out_base + dg * DG_STRIDE, PIECE)],
                ssem,
            )
            for dg in range(8)
        ]

    def drain_stage(g, stg, ssem):
        n = g // Q_PER_ROW
        lq = g % Q_PER_ROW
        out_base = n * N_STRIDE + lq * PIECE
        for dg in range(8):
            pltpu.make_async_copy(
                stg.at[pl.ds(dg * PIECE, PIECE)],
                out_hbm.at[pl.ds(out_base + dg * DG_STRIDE, PIECE)],
                ssem,
            ).wait()

    def pair_body(p, carry):
        g0 = wid * CH_PER_W + 2 * p
        g1 = g0 + 1

        cps0 = prepare(g0, x_v0, idx_v0, rows_v0, gsem0)
        cps1 = prepare(g1, x_v1, idx_v1, rows_v1, gsem1)

        for cp in cps0:
            cp.wait()

        @pl.when(p > 0)
        def _():
            drain_stage(g0, stg0, ssem0)

        transpose_and_emit(g0, rows_v0, stg0, ssem0)

        for cp in cps1:
            cp.wait()

        @pl.when(p > 0)
        def _():
            drain_stage(g1, stg1, ssem1)

        transpose_and_emit(g1, rows_v1, stg1, ssem1)
        return carry

    lax.fori_loop(0, PAIRS, pair_body, 0)

    g_last0 = wid * CH_PER_W + CH_PER_W - 2
    g_last1 = wid * CH_PER_W + CH_PER_W - 1
    drain_stage(g_last0, stg0, ssem0)
    drain_stage(g_last1, stg1, ssem1)


def _gather(table, x_flat):
    mesh = plsc.VectorSubcoreMesh(core_axis_name="c", subcore_axis_name="s")
    k = functools.partial(
        pl.kernel,
        mesh=mesh,
        out_type=jax.ShapeDtypeStruct((TOK * D,), jnp.float32),
        scratch_types=[
            pltpu.VMEM((CHUNK,), jnp.int32),
            pltpu.VMEM((CHUNK,), jnp.int32),
            pltpu.VMEM((IDX_ROWS, 128), jnp.int32),
            pltpu.VMEM((IDX_ROWS, 128), jnp.int32),
            pltpu.VMEM((CHUNK, D), jnp.float32),
            pltpu.VMEM((CHUNK, D), jnp.float32),
            pltpu.VMEM((STAGE,), jnp.float32),
            pltpu.VMEM((STAGE,), jnp.float32),
            pltpu.SemaphoreType.DMA,
            pltpu.SemaphoreType.DMA,
            pltpu.SemaphoreType.DMA,
            pltpu.SemaphoreType.DMA,
        ],
        compiler_params=pltpu.CompilerParams(
            use_tc_tiling_on_sc=False, needs_layout_passes=False
        ),
    )(_gather_kernel)
    return k(table, x_flat)


def kernel(x, W_emb, W_q, pos_enc):
    table = _build_table(W_emb, W_q, pos_enc)
    x_flat = x.reshape(TOK).astype(jnp.int32)
    out1 = _gather(table, x_flat)
    out6 = out1.reshape(B, N, 8, L // 128, 8, 128)
    return out6.transpose(0, 1, 3, 5, 2, 4).reshape(B, N, L, D)


# speedup vs baseline: 2.1469x; 1.1574x over previous
import functools

import jax
import jax.numpy as jnp
from jax import lax
from jax.experimental import pallas as pl
from jax.experimental.pallas import tpu as pltpu
from jax.experimental.pallas import tpu_sc as plsc

B, N, L, D = 1, 512, 1024, 64
V = 21
TOK = B * N * L
NC, NS = 2, 16
NW = NC * NS
CHUNK = 256
CH_PER_W = TOK // (NW * CHUNK)
IDX_ROWS = CHUNK // 128
Q_PER_ROW = L // CHUNK
STAGE = CHUNK * D
N_STRIDE = D * L
DG_STRIDE = 8 * L
PIECE = 2 * 8 * 128
DEPTH = 4


def _table_body(we_ref, wq_ref, pe_ref, out_ref):
    pe = pe_ref[...]
    we = we_ref[...]
    for q in range(2):
        wq = wq_ref[q]
        out_ref[q] = pe[:, None, :] + we[None, :, :] + wq[None, None, :]


def _build_table(W_emb, W_q, pos_enc):
    t = pl.pallas_call(
        _table_body,
        out_shape=jax.ShapeDtypeStruct((2, L, V, D), jnp.float32),
    )(W_emb, W_q, pos_enc)
    return t.reshape(2 * L * V, D)


def _gather_kernel(
    table_hbm, x_hbm, out_hbm,
    x_v0, x_v1, x_v2, x_v3,
    idx_v0, idx_v1, idx_v2, idx_v3,
    rows_v0, rows_v1, rows_v2, rows_v3,
    stg0, stg1,
    xsem0, xsem1, xsem2, xsem3,
    gsem0, gsem1, gsem2, gsem3,
    ssem0, ssem1,
):
    XB = [x_v0, x_v1, x_v2, x_v3]
    IB = [idx_v0, idx_v1, idx_v2, idx_v3]
    RB = [rows_v0, rows_v1, rows_v2, rows_v3]
    SG = [stg0, stg1]
    XS = [xsem0, xsem1, xsem2, xsem3]
    GS = [gsem0, gsem1, gsem2, gsem3]
    SS = [ssem0, ssem1]

    wid = lax.axis_index("s") * NC + lax.axis_index("c")
    iota16 = lax.iota(jnp.int32, 16)
    c0 = wid * CH_PER_W

    def fire_x(g, x_v, xsem):
        pltpu.async_copy(x_hbm.at[pl.ds(g * CHUNK, CHUNK)], x_v, xsem)

    def build_and_fire(g, x_v, idx_v, rows_v, xsem, gsem):
        pltpu.make_async_copy(
            x_hbm.at[pl.ds(g * CHUNK, CHUNK)], x_v, xsem
        ).wait()
        l_base = (g % Q_PER_ROW) * CHUNK
        qoff = jnp.where(g >= Q_PER_ROW, L * V, 0).astype(jnp.int32)

        for j in range(IDX_ROWS):
            def idx_body(i, _):
                t = j * 128 + i * 16
                xv = x_v[pl.ds(t, 16)]
                lv = iota16 + (l_base + t)
                idx_v[j, pl.ds(i * 16, 16)] = xv + lv * V + qoff
                return 0

            lax.fori_loop(0, 128 // 16, idx_body, 0)

        for j in range(IDX_ROWS):
            pltpu.async_copy(
                table_hbm.at[idx_v.at[j]],
                rows_v.at[pl.ds(j * 128, 128)],
                gsem,
            )

    def wait_gathers(idx_v, rows_v, gsem):
        for j in range(IDX_ROWS):
            pltpu.make_async_copy(
                table_hbm.at[idx_v.at[j]],
                rows_v.at[pl.ds(j * 128, 128)],
                gsem,
            ).wait()

    w_pats = []
    s_pats = []
    for s in range(16):
        w = (iota16 + s) & 15
        w_pats.append(w)
        s_pats.append(((w >> 3) << 11) + ((w & 7) << 7) + iota16)

    def transpose_and_emit(g, rows_v, stg, ssem):
        n = g // Q_PER_ROW
        lq = g % Q_PER_ROW

        def tb_body(tb, _):
            t0 = tb * 16
            rowidx = iota16 + t0
            tconst = (t0 // 128) * 1024 + (t0 % 128)
            for jd in range(4):
                d0 = jd * 16
                bsplat = jnp.zeros((16,), jnp.int32) + (tconst + (d0 >> 3) * PIECE)
                for h in range(2):
                    vals = [
                        plsc.load_gather(rows_v, [rowidx, w_pats[h * 8 + s] + d0])
                        for s in range(8)
                    ]
                    for s in range(8):
                        plsc.store_scatter(stg, [s_pats[h * 8 + s] + bsplat], vals[s])
            return 0

        lax.fori_loop(0, CHUNK // 16, tb_body, 0)

        out_base = n * N_STRIDE + lq * PIECE
        for dg in range(8):
            pltpu.async_copy(
                stg.at[pl.ds(dg * PIECE, PIECE)],
                out_hbm.at[pl.ds(out_base + dg * DG_STRIDE, PIECE)],
                ssem,
            )

    def drain_stage(g, stg, ssem):
        n = g // Q_PER_ROW
        lq = g % Q_PER_ROW
        out_base = n * N_STRIDE + lq * PIECE
        for dg in range(8):
            pltpu.make_async_copy(
                stg.at[pl.ds(dg * PIECE, PIECE)],
                out_hbm.at[pl.ds(out_base + dg * DG_STRIDE, PIECE)],
                ssem,
            ).wait()

    for k in range(DEPTH):
        fire_x(c0 + k, XB[k], XS[k])
    for k in range(DEPTH):
        build_and_fire(c0 + k, XB[k], IB[k], RB[k], XS[k], GS[k])

    def quad_body(p, carry):
        for k in range(DEPTH):
            c = c0 + DEPTH * p + k
            wait_gathers(IB[k], RB[k], GS[k])
            if k >= 2:
                drain_stage(c - 2, SG[k % 2], SS[k % 2])
            else:
                @pl.when(p > 0)
                def _(c=c, k=k):
                    drain_stage(c - 2, SG[k % 2], SS[k % 2])
            transpose_and_emit(c, RB[k], SG[k % 2], SS[k % 2])

            @pl.when(p < CH_PER_W // DEPTH - 1)
            def _(c=c, k=k):
                fire_x(c + DEPTH, XB[k], XS[k])
                build_and_fire(c + DEPTH, XB[k], IB[k], RB[k], XS[k], GS[k])
        return carry

    lax.fori_loop(0, CH_PER_W // DEPTH, quad_body, 0)

    drain_stage(c0 + CH_PER_W - 2, SG[0], SS[0])
    drain_stage(c0 + CH_PER_W - 1, SG[1], SS[1])


def _gather(table, x_flat):
    mesh = plsc.VectorSubcoreMesh(core_axis_name="c", subcore_axis_name="s")
    k = functools.partial(
        pl.kernel,
        mesh=mesh,
        out_type=jax.ShapeDtypeStruct((TOK * D,), jnp.float32),
        scratch_types=(
            [pltpu.VMEM((CHUNK,), jnp.int32)] * 4
            + [pltpu.VMEM((IDX_ROWS, 128), jnp.int32)] * 4
            + [pltpu.VMEM((CHUNK, D), jnp.float32)] * 4
            + [pltpu.VMEM((STAGE,), jnp.float32)] * 2
            + [pltpu.SemaphoreType.DMA] * 10
        ),
        compiler_params=pltpu.CompilerParams(
            use_tc_tiling_on_sc=False, needs_layout_passes=False
        ),
    )(_gather_kernel)
    return k(table, x_flat)


def kernel(x, W_emb, W_q, pos_enc):
    table = _build_table(W_emb, W_q, pos_enc)
    x_flat = x.reshape(TOK).astype(jnp.int32)
    out1 = _gather(table, x_flat)
    out6 = out1.reshape(B, N, 8, L // 128, 8, 128)
    return out6.transpose(0, 1, 3, 5, 2, 4).reshape(B, N, L, D)
